# baseline (speedup 1.0000x reference)
import functools

import jax
import jax.numpy as jnp
from jax import lax
from jax.experimental import pallas as pl
from jax.experimental.pallas import tpu as pltpu
from jax.experimental.pallas import tpu_sc as plsc

N = 10000
NUM_WORKERS = 32
PAIRS_PER_WORKER = 157
NPAD = NUM_WORKERS * PAIRS_PER_WORKER * 2
DEPTH = 8
GROUP = 8
PAD_BLKS = 16
PADL = PAD_BLKS * 16
NBLK_TOT = PAD_BLKS + (N // 16) + PAD_BLKS
RIGHT_MAX = NBLK_TOT - GROUP
COORD_LEN = NBLK_TOT * 16 + 16
PAD_COORD = 1.0e5
ROW_OUT = DEPTH * 16


def _sc_topk(xs, ys, zs):
    mesh = plsc.VectorSubcoreMesh(
        core_axis_name="c", subcore_axis_name="s",
        num_cores=2, num_subcores=16,
    )

    @functools.partial(
        pl.kernel,
        out_type=jax.ShapeDtypeStruct(
            (PAIRS_PER_WORKER, NUM_WORKERS, 2, 1, ROW_OUT), jnp.float32),
        mesh=mesh,
        scratch_types=[
            pltpu.VMEM((COORD_LEN,), jnp.float32),
            pltpu.VMEM((COORD_LEN,), jnp.float32),
            pltpu.VMEM((COORD_LEN,), jnp.float32),
            pltpu.VMEM((PAIRS_PER_WORKER, 2, 1, ROW_OUT), jnp.float32),
        ],
        compiler_params=pltpu.CompilerParams(needs_layout_passes=False),
    )
    def topk_kernel(xs_hbm, ys_hbm, zs_hbm, out_hbm, xv, yv, zv, res):
        wid = lax.axis_index("s") * 2 + lax.axis_index("c")
        pltpu.sync_copy(xs_hbm, xv)
        pltpu.sync_copy(ys_hbm, yv)
        pltpu.sync_copy(zs_hbm, zv)

        inf16 = jnp.full((16,), jnp.inf, jnp.float32)
        lane01 = lax.iota(jnp.int32, 16) < 2

        def pair_body(k, carry):
            srow_a = jnp.minimum(2 * (wid + 32 * k), N - 2)
            crow = srow_a + PADL
            wx = xv[pl.ds(crow, 16)]
            wy = yv[pl.ds(crow, 16)]
            wz = zv[pl.ds(crow, 16)]
            xa, xb = wx[0], wx[1]
            ya, yb = wy[0], wy[1]
            za, zb = wz[0], wz[1]
            own = crow // 16
            xv[pl.ds(crow, 16)] = jnp.where(lane01, PAD_COORD, wx)

            def proc_group(p, best):
                ba, bb = best[:DEPTH], best[DEPTH:]
                for j in range(GROUP):
                    off = (p + j) * 16
                    cx = xv[pl.ds(off, 16)]
                    cy = yv[pl.ds(off, 16)]
                    cz = zv[pl.ds(off, 16)]
                    dxa = cx - xa
                    dya = cy - ya
                    dza = cz - za
                    ca = dxa * dxa + dya * dya + dza * dza
                    dxb = cx - xb
                    dyb = cy - yb
                    dzb = cz - zb
                    cb = dxb * dxb + dyb * dyb + dzb * dzb
                    na, nb = [], []
                    for bk in ba:
                        na.append(jnp.minimum(bk, ca))
                        ca = jnp.maximum(bk, ca)
                    for bk in bb:
                        nb.append(jnp.minimum(bk, cb))
                        cb = jnp.maximum(bk, cb)
                    ba, bb = na, nb
                return list(ba) + list(bb)

            def bound(m_prev, b0, b1):
                lo16 = jnp.sort(jnp.minimum(
                    jnp.sort(b0), lax.rev(jnp.sort(b1), (0,))))
                return jnp.minimum(m_prev, lo16[8])

            def bounds(st, best):
                ma = bound(st[1], best[0], best[1])
                mb = bound(st[2], best[DEPTH], best[DEPTH + 1])
                return ma, mb

            def rcond(st):
                p = st[0]
                edge = xv[pl.ds(p * 16, 16)][0] - xa
                return (p <= RIGHT_MAX) & (edge * edge <= jnp.maximum(
                    st[1], st[2]))

            def rbody(st):
                p = st[0]
                best = proc_group(p, list(st[3:]))
                ma, mb = bounds(st, best)
                return (p + GROUP, ma, mb) + tuple(best)

            st = lax.while_loop(
                rcond, rbody, (own, jnp.inf, jnp.inf) + (inf16,) * (2 * DEPTH))
            ma_cur, mb_cur = st[1], st[2]
            best0 = st[3:]

            def lcond(st):
                p = st[0]
                edge = xv[pl.ds(jnp.maximum(p * 16 + (GROUP * 16 - 1), 0),
                                16)][0] - xa
                return (p >= 0) & (edge * edge <= jnp.maximum(st[1], st[2]))

            def lbody(st):
                p = st[0]
                best = proc_group(p, list(st[3:]))
                ma, mb = bounds(st, best)
                return (p - GROUP, ma, mb) + tuple(best)

            st = lax.while_loop(
                lcond, lbody, (own - GROUP, ma_cur, mb_cur) + tuple(best0))

            xv[pl.ds(crow, 16)] = wx
            for j in range(DEPTH):
                res[k, 0, 0, pl.ds(j * 16, 16)] = st[3 + j]
                res[k, 1, 0, pl.ds(j * 16, 16)] = st[3 + DEPTH + j]
            return carry

        lax.fori_loop(0, PAIRS_PER_WORKER, pair_body, 0)
        pltpu.sync_copy(res, out_hbm.at[:, wid])

    return topk_kernel(xs, ys, zs)


def _tc_loss(top, s_aligned):
    def body(top_ref, s_ref, out_ref):
        d = top_ref[:]
        total = jnp.zeros((NPAD, 1), jnp.float32)
        for _ in range(DEPTH):
            m = jnp.min(d, axis=1, keepdims=True)
            total = total + jnp.sqrt(m)
            d = jnp.where(d == m, jnp.inf, d)
        elen = total * (1.0 / DEPTH)
        diff = s_ref[:] - elen
        sq = diff * diff
        t = lax.broadcasted_iota(jnp.int32, (NPAD, 1), 0)
        sq = jnp.where(t < N, sq, 0.0)
        out_ref[0, 0] = jnp.sum(sq) / N

    return pl.pallas_call(
        body,
        out_shape=jax.ShapeDtypeStruct((1, 1), jnp.float32),
        out_specs=pl.BlockSpec(memory_space=pltpu.SMEM),
    )(top, s_aligned)


def kernel(xyz_canon, scales):
    xc, yc, zc, s_sorted = lax.sort(
        (xyz_canon[:, 0], xyz_canon[:, 1], xyz_canon[:, 2], scales[:, 0]),
        num_keys=1)

    padl = jnp.full((PADL,), -PAD_COORD, jnp.float32)
    padr = jnp.full((COORD_LEN - PADL - N,), PAD_COORD, jnp.float32)
    xs = jnp.concatenate([padl, xc, padr])
    ys = jnp.concatenate([padl, yc, padr])
    zs = jnp.concatenate([padl, zc, padr])
    s_pad = jnp.concatenate(
        [s_sorted, jnp.zeros((NPAD - N,), jnp.float32)])[:, None]

    top = _sc_topk(xs, ys, zs).reshape(NPAD, ROW_OUT)
    loss = _tc_loss(top, s_pad)
    return loss[0, 0]


# speedup vs baseline: 1.0019x; 1.0019x over previous
import functools

import jax
import jax.numpy as jnp
from jax import lax
from jax.experimental import pallas as pl
from jax.experimental.pallas import tpu as pltpu
from jax.experimental.pallas import tpu_sc as plsc

N = 10000
NUM_WORKERS = 32
PAIRS_PER_WORKER = 157
NPAD = NUM_WORKERS * PAIRS_PER_WORKER * 2
DEPTH = 8
GROUP = 16
PAD_BLKS = 16
PADL = PAD_BLKS * 16
NBLK_TOT = PAD_BLKS + (N // 16) + PAD_BLKS
RIGHT_MAX = NBLK_TOT - GROUP
COORD_LEN = NBLK_TOT * 16 + 16
PAD_COORD = 1.0e5
ROW_OUT = DEPTH * 16


def _sc_topk(xs, ys, zs):
    mesh = plsc.VectorSubcoreMesh(
        core_axis_name="c", subcore_axis_name="s",
        num_cores=2, num_subcores=16,
    )

    @functools.partial(
        pl.kernel,
        out_type=jax.ShapeDtypeStruct(
            (PAIRS_PER_WORKER, NUM_WORKERS, 2, 1, ROW_OUT), jnp.float32),
        mesh=mesh,
        scratch_types=[
            pltpu.VMEM((COORD_LEN,), jnp.float32),
            pltpu.VMEM((COORD_LEN,), jnp.float32),
            pltpu.VMEM((COORD_LEN,), jnp.float32),
            pltpu.VMEM((PAIRS_PER_WORKER, 2, 1, ROW_OUT), jnp.float32),
        ],
        compiler_params=pltpu.CompilerParams(needs_layout_passes=False),
    )
    def topk_kernel(xs_hbm, ys_hbm, zs_hbm, out_hbm, xv, yv, zv, res):
        wid = lax.axis_index("s") * 2 + lax.axis_index("c")
        pltpu.sync_copy(xs_hbm, xv)
        pltpu.sync_copy(ys_hbm, yv)
        pltpu.sync_copy(zs_hbm, zv)

        inf16 = jnp.full((16,), jnp.inf, jnp.float32)
        lane01 = lax.iota(jnp.int32, 16) < 2

        def pair_body(k, carry):
            srow_a = jnp.minimum(2 * (wid + 32 * k), N - 2)
            crow = srow_a + PADL
            wx = xv[pl.ds(crow, 16)]
            wy = yv[pl.ds(crow, 16)]
            wz = zv[pl.ds(crow, 16)]
            xa, xb = wx[0], wx[1]
            ya, yb = wy[0], wy[1]
            za, zb = wz[0], wz[1]
            own = crow // 16
            xv[pl.ds(crow, 16)] = jnp.where(lane01, PAD_COORD, wx)

            def proc_group(p, best):
                ba, bb = best[:DEPTH], best[DEPTH:]
                for j in range(GROUP):
                    off = (p + j) * 16
                    cx = xv[pl.ds(off, 16)]
                    cy = yv[pl.ds(off, 16)]
                    cz = zv[pl.ds(off, 16)]
                    dxa = cx - xa
                    dya = cy - ya
                    dza = cz - za
                    ca = dxa * dxa + dya * dya + dza * dza
                    dxb = cx - xb
                    dyb = cy - yb
                    dzb = cz - zb
                    cb = dxb * dxb + dyb * dyb + dzb * dzb
                    na, nb = [], []
                    for bk in ba:
                        na.append(jnp.minimum(bk, ca))
                        ca = jnp.maximum(bk, ca)
                    for bk in bb:
                        nb.append(jnp.minimum(bk, cb))
                        cb = jnp.maximum(bk, cb)
                    ba, bb = na, nb
                return list(ba) + list(bb)

            def bound(m_prev, b0, b1):
                lo16 = jnp.sort(jnp.minimum(
                    jnp.sort(b0), lax.rev(jnp.sort(b1), (0,))))
                return jnp.minimum(m_prev, lo16[8])

            def bounds(st, best):
                ma = bound(st[1], best[0], best[1])
                mb = bound(st[2], best[DEPTH], best[DEPTH + 1])
                return ma, mb

            def rcond(st):
                p = st[0]
                edge = xv[pl.ds(p * 16, 16)][0] - xa
                return (p <= RIGHT_MAX) & (edge * edge <= jnp.maximum(
                    st[1], st[2]))

            def rbody(st):
                p = st[0]
                best = proc_group(p, list(st[3:]))
                ma, mb = bounds(st, best)
                return (p + GROUP, ma, mb) + tuple(best)

            st = lax.while_loop(
                rcond, rbody, (own, jnp.inf, jnp.inf) + (inf16,) * (2 * DEPTH))
            ma_cur, mb_cur = st[1], st[2]
            best0 = st[3:]

            def lcond(st):
                p = st[0]
                edge = xv[pl.ds(jnp.maximum(p * 16 + (GROUP * 16 - 1), 0),
                                16)][0] - xa
                return (p >= 0) & (edge * edge <= jnp.maximum(st[1], st[2]))

            def lbody(st):
                p = st[0]
                best = proc_group(p, list(st[3:]))
                ma, mb = bounds(st, best)
                return (p - GROUP, ma, mb) + tuple(best)

            st = lax.while_loop(
                lcond, lbody, (own - GROUP, ma_cur, mb_cur) + tuple(best0))

            xv[pl.ds(crow, 16)] = wx
            for j in range(DEPTH):
                res[k, 0, 0, pl.ds(j * 16, 16)] = st[3 + j]
                res[k, 1, 0, pl.ds(j * 16, 16)] = st[3 + DEPTH + j]
            return carry

        lax.fori_loop(0, PAIRS_PER_WORKER, pair_body, 0)
        pltpu.sync_copy(res, out_hbm.at[:, wid])

    return topk_kernel(xs, ys, zs)


def _tc_loss(top, s_aligned):
    def body(top_ref, s_ref, out_ref):
        d = top_ref[:]
        total = jnp.zeros((NPAD, 1), jnp.float32)
        for _ in range(DEPTH):
            m = jnp.min(d, axis=1, keepdims=True)
            total = total + jnp.sqrt(m)
            d = jnp.where(d == m, jnp.inf, d)
        elen = total * (1.0 / DEPTH)
        diff = s_ref[:] - elen
        sq = diff * diff
        t = lax.broadcasted_iota(jnp.int32, (NPAD, 1), 0)
        sq = jnp.where(t < N, sq, 0.0)
        out_ref[0, 0] = jnp.sum(sq) / N

    return pl.pallas_call(
        body,
        out_shape=jax.ShapeDtypeStruct((1, 1), jnp.float32),
        out_specs=pl.BlockSpec(memory_space=pltpu.SMEM),
    )(top, s_aligned)


def kernel(xyz_canon, scales):
    xc, yc, zc, s_sorted = lax.sort(
        (xyz_canon[:, 0], xyz_canon[:, 1], xyz_canon[:, 2], scales[:, 0]),
        num_keys=1)

    padl = jnp.full((PADL,), -PAD_COORD, jnp.float32)
    padr = jnp.full((COORD_LEN - PADL - N,), PAD_COORD, jnp.float32)
    xs = jnp.concatenate([padl, xc, padr])
    ys = jnp.concatenate([padl, yc, padr])
    zs = jnp.concatenate([padl, zc, padr])
    s_pad = jnp.concatenate(
        [s_sorted, jnp.zeros((NPAD - N,), jnp.float32)])[:, None]

    top = _sc_topk(xs, ys, zs).reshape(NPAD, ROW_OUT)
    loss = _tc_loss(top, s_pad)
    return loss[0, 0]
